# baseline (speedup 1.0000x reference)
import math

import jax
import jax.numpy as jnp
from jax import lax
from jax.experimental import pallas as pl
from jax.experimental.pallas import tpu as pltpu
from jax.experimental.pallas import tpu_sc as plsc

VOCAB = 100000
D_MODEL = 64
MAX_LEN = 512
BATCH = 1024
SEQ = 512

LANES = 16
NUM_CORES = 2
NUM_SUBCORES = 16
NUM_WORKERS = NUM_CORES * NUM_SUBCORES

CHUNK = 128
ROWS = BATCH * SEQ
NCHUNKS = ROWS // CHUNK
CHUNKS_PER_W = NCHUNKS // NUM_WORKERS
NBUF = 8
VPR = D_MODEL // LANES


def _positional_encoding():
    position = jnp.arange(0, MAX_LEN, dtype=jnp.float32)[:, None]
    div_term = jnp.exp(
        jnp.arange(0, D_MODEL, 2, dtype=jnp.float32) * (-math.log(10000.0) / D_MODEL)
    )
    pe = jnp.zeros((MAX_LEN, D_MODEL), dtype=jnp.float32)
    pe = pe.at[:, 0::2].set(jnp.sin(position * div_term))
    pe = pe.at[:, 1::2].set(jnp.cos(position * div_term))
    return pe


def _body(src_ids, tgt_ids, table, pe_hbm, src_out, tgt_out,
          ids_v, pe_v, bufs, gsems, wsems):
    c = lax.axis_index("c")
    s = lax.axis_index("s")
    wid = s * NUM_CORES + c
    chunk_base = wid * CHUNKS_PER_W

    pltpu.sync_copy(pe_hbm, pe_v)

    def add_pe(buf, pe_off):
        @plsc.parallel_loop(0, CHUNK, unroll=8)
        def _(r):
            for k in range(VPR):
                sl = pl.ds(k * LANES, LANES)
                plsc.addupdate(buf.at[r, sl], pe_v[pe_off + r, sl])

    for ids_hbm, out_hbm in ((src_ids, src_out), (tgt_ids, tgt_out)):
        pltpu.sync_copy(ids_hbm.at[pl.ds(chunk_base, CHUNKS_PER_W)], ids_v)

        for b in range(NBUF - 1):
            pltpu.async_copy(table.at[ids_v.at[b]], bufs[b], gsems[b])

        @pl.loop(0, CHUNKS_PER_W // NBUF)
        def _(j):
            for k in range(NBUF):
                t = NBUF * j + k
                p = k
                q = (k - 1) % NBUF
                tg = t + NBUF - 1

                def refill(j=j, t=t, q=q, tg=tg, k=k):
                    if k == 0:
                        @pl.when(j >= 1)
                        def _():
                            pltpu.make_async_copy(
                                bufs[q], out_hbm.at[pl.ds(0, CHUNK)], wsems[q]
                            ).wait()
                    else:
                        pltpu.make_async_copy(
                            bufs[q], out_hbm.at[pl.ds(0, CHUNK)], wsems[q]
                        ).wait()
                    pltpu.async_copy(table.at[ids_v.at[tg]], bufs[q], gsems[q])

                max_j = (CHUNKS_PER_W - NBUF - k) // NBUF
                if max_j >= CHUNKS_PER_W // NBUF - 1:
                    refill()
                else:
                    @pl.when(j <= max_j)
                    def _():
                        refill()

                pltpu.make_async_copy(
                    table.at[ids_v.at[t]], bufs[p], gsems[p]
                ).wait()
                add_pe(bufs[p], (k % (SEQ // CHUNK)) * CHUNK)
                row_base = (chunk_base + t) * CHUNK
                pltpu.async_copy(
                    bufs[p], out_hbm.at[pl.ds(row_base, CHUNK)], wsems[p]
                )

        for b in range(NBUF):
            pltpu.make_async_copy(
                bufs[b], out_hbm.at[pl.ds(0, CHUNK)], wsems[b]
            ).wait()


def kernel(input_ids, decoder_input_ids, embedding):
    pe = _positional_encoding()
    src_ids = input_ids.reshape(NCHUNKS, CHUNK).astype(jnp.int32)
    tgt_ids = decoder_input_ids.reshape(NCHUNKS, CHUNK).astype(jnp.int32)
    out_t = jax.ShapeDtypeStruct((ROWS, D_MODEL), jnp.float32)

    f = pl.kernel(
        _body,
        out_type=(out_t, out_t),
        mesh=plsc.VectorSubcoreMesh(core_axis_name="c", subcore_axis_name="s"),
        compiler_params=pltpu.CompilerParams(use_tc_tiling_on_sc=False),
        scratch_types=[
            pltpu.VMEM((CHUNKS_PER_W, CHUNK), jnp.int32),
            pltpu.VMEM((MAX_LEN, D_MODEL), jnp.float32),
            [pltpu.VMEM((CHUNK, D_MODEL), jnp.float32) for _ in range(NBUF)],
            [pltpu.SemaphoreType.DMA for _ in range(NBUF)],
            [pltpu.SemaphoreType.DMA for _ in range(NBUF)],
        ],
    )
    src_flat, tgt_flat = f(src_ids, tgt_ids, embedding, pe)
    return (src_flat.reshape(BATCH, SEQ, D_MODEL),
            tgt_flat.reshape(BATCH, SEQ, D_MODEL))


# speedup vs baseline: 1.6361x; 1.6361x over previous
import math

import jax
import jax.numpy as jnp
from jax import lax
from jax.experimental import pallas as pl
from jax.experimental.pallas import tpu as pltpu
from jax.experimental.pallas import tpu_sc as plsc

VOCAB = 100000
D_MODEL = 64
MAX_LEN = 512
BATCH = 1024
SEQ = 512

LANES = 16
NUM_CORES = 2
NUM_SUBCORES = 16
NUM_WORKERS = NUM_CORES * NUM_SUBCORES

CHUNK = 128
ROWS = BATCH * SEQ
NCHUNKS = ROWS // CHUNK
CHUNKS_PER_W = NCHUNKS // NUM_WORKERS
CPS = SEQ // CHUNK
NBUF = 4
DGR = D_MODEL // LANES
TPAD = CHUNK + 1


def _positional_encoding():
    position = jnp.arange(0, MAX_LEN, dtype=jnp.float32)[:, None]
    div_term = jnp.exp(
        jnp.arange(0, D_MODEL, 2, dtype=jnp.float32) * (-math.log(10000.0) / D_MODEL)
    )
    pe = jnp.zeros((MAX_LEN, D_MODEL), dtype=jnp.float32)
    pe = pe.at[:, 0::2].set(jnp.sin(position * div_term))
    pe = pe.at[:, 1::2].set(jnp.cos(position * div_term))
    return pe


def _body(ids_hbm, table, pe_hbm, out_hbm,
          ids_v, pe_v, bufs, tbufs, gsems, wsems):
    c = lax.axis_index("c")
    s = lax.axis_index("s")
    wid = s * NUM_CORES + c
    chunk_base = wid * CHUNKS_PER_W

    pltpu.sync_copy(pe_hbm, pe_v)

    d_ids = [lax.iota(jnp.int32, LANES) + m * LANES for m in range(DGR)]

    def transpose_add(src, dst, s0):
        @plsc.parallel_loop(0, CHUNK, unroll=8)
        def _(si):
            si_vec = jnp.full((LANES,), si, dtype=jnp.int32)
            for m in range(DGR):
                sl = pl.ds(m * LANES, LANES)
                v = src[si, sl] + pe_v[s0 + si, sl]
                plsc.store_scatter(dst, [d_ids[m], si_vec], v)

    if True:
        pltpu.sync_copy(ids_hbm.at[pl.ds(chunk_base, CHUNKS_PER_W)], ids_v)

        for b in range(NBUF - 1):
            pltpu.async_copy(table.at[ids_v.at[b]], bufs[b], gsems[b])

        @pl.loop(0, CHUNKS_PER_W // NBUF)
        def _(j):
            for k in range(NBUF):
                t = NBUF * j + k
                p = k
                q = (k + NBUF - 1) % NBUF
                tg = t + NBUF - 1

                def refill(q=q, tg=tg):
                    pltpu.async_copy(table.at[ids_v.at[tg]], bufs[q], gsems[q])

                max_j = (CHUNKS_PER_W - NBUF - k) // NBUF
                if max_j >= CHUNKS_PER_W // NBUF - 1:
                    refill()
                else:
                    @pl.when(j <= max_j)
                    def _():
                        refill()

                pltpu.make_async_copy(
                    table.at[ids_v.at[t]], bufs[p], gsems[p]
                ).wait()

                @pl.when(j >= 1)
                def _():
                    pltpu.make_async_copy(
                        tbufs[p].at[:, pl.ds(0, CHUNK)],
                        out_hbm.at[0, :, pl.ds(0, CHUNK)], wsems[p]
                    ).wait()

                s0 = (k % CPS) * CHUNK
                transpose_add(bufs[p], tbufs[p], s0)
                b_idx = chunk_base // CPS + (NBUF // CPS) * j + k // CPS
                pltpu.async_copy(
                    tbufs[p].at[:, pl.ds(0, CHUNK)],
                    out_hbm.at[b_idx, :, pl.ds(s0, CHUNK)], wsems[p]
                )

        for b in range(NBUF):
            pltpu.make_async_copy(
                tbufs[b].at[:, pl.ds(0, CHUNK)],
                out_hbm.at[0, :, pl.ds(0, CHUNK)], wsems[b]
            ).wait()


def kernel(input_ids, decoder_input_ids, embedding):
    pe_t = _positional_encoding()
    src_ids = input_ids.reshape(NCHUNKS, CHUNK).astype(jnp.int32)
    tgt_ids = decoder_input_ids.reshape(NCHUNKS, CHUNK).astype(jnp.int32)
    out_t = jax.ShapeDtypeStruct((BATCH, D_MODEL, SEQ), jnp.float32)

    f = pl.kernel(
        _body,
        out_type=out_t,
        mesh=plsc.VectorSubcoreMesh(core_axis_name="c", subcore_axis_name="s"),
        compiler_params=pltpu.CompilerParams(
            use_tc_tiling_on_sc=False, needs_layout_passes=False
        ),
        scratch_types=[
            pltpu.VMEM((CHUNKS_PER_W, CHUNK), jnp.int32),
            pltpu.VMEM((MAX_LEN, D_MODEL), jnp.float32),
            [pltpu.VMEM((CHUNK, D_MODEL), jnp.float32) for _ in range(NBUF)],
            [pltpu.VMEM((D_MODEL, TPAD), jnp.float32) for _ in range(NBUF)],
            [pltpu.SemaphoreType.DMA for _ in range(NBUF)],
            [pltpu.SemaphoreType.DMA for _ in range(NBUF)],
        ],
    )
    src_t = f(src_ids, embedding, pe_t)
    tgt_t = f(tgt_ids, embedding, pe_t)
    return (jnp.swapaxes(src_t, 1, 2), jnp.swapaxes(tgt_t, 1, 2))
